# baseline (speedup 1.0000x reference)
import functools
import math

import jax
import jax.numpy as jnp
from jax import lax
from jax.experimental import pallas as pl
from jax.experimental.pallas import tpu as pltpu
from jax.experimental.pallas import tpu_sc as plsc

_B, _L, _D, _V = 4096, 200, 64, 1000000
_NC, _NS = 2, 16
_NW = _NC * _NS
_SEQ_W = _B // _NW
_ROWS_W = _SEQ_W * _L
_C = 2 * _L
_NG = _ROWS_W // _C
_IPG = 100
_PPC = _C // _IPG
_IDX_ROWS = _ROWS_W // _IPG


def _pe_body(out_ref):
    row = lax.broadcasted_iota(jnp.int32, (_L, _D), 0).astype(jnp.float32)
    col = lax.broadcasted_iota(jnp.int32, (_L, _D), 1)
    expo = (col // 2).astype(jnp.float32) * (2.0 / _D)
    denom = jnp.exp(expo * math.log(10000.0))
    angle = row / denom
    out_ref[...] = jnp.where(col % 2 == 0, jnp.sin(angle), jnp.cos(angle))


def _make_pe():
    return pl.pallas_call(
        _pe_body,
        out_shape=jax.ShapeDtypeStruct((_L, _D), jnp.float32),
    )()


_sc_mesh = plsc.VectorSubcoreMesh(core_axis_name="c", subcore_axis_name="s")


@functools.partial(
    pl.kernel,
    out_type=jax.ShapeDtypeStruct((_B * _L, _D), jnp.float32),
    mesh=_sc_mesh,
    scratch_types=[
        pltpu.VMEM((_IDX_ROWS, _IPG), jnp.int32),
        pltpu.VMEM((2, _C, _D), jnp.float32),
        pltpu.VMEM_SHARED((_C, _D), jnp.float32),
        pltpu.SemaphoreType.DMA((2,)),
        pltpu.SemaphoreType.DMA((2,)),
    ],
    compiler_params=pltpu.CompilerParams(use_tc_tiling_on_sc=False),
)
def _sc_embed(tok_hbm, pe_hbm, table_hbm, out_hbm, idx_v, rows_v, pe2_sh,
              gsem, osem):
    sid = lax.axis_index("s")
    wid = sid * _NC + lax.axis_index("c")
    row0 = wid * _ROWS_W
    irow0 = wid * _IDX_ROWS

    pltpu.sync_copy(tok_hbm.at[pl.ds(irow0, _IDX_ROWS)], idx_v)

    @pl.when(sid == 0)
    def _():
        pltpu.sync_copy(pe_hbm, pe2_sh.at[pl.ds(0, _L)])
        pltpu.sync_copy(pe_hbm, pe2_sh.at[pl.ds(_L, _L)])
    plsc.subcore_barrier()

    def prefill(b):
        pltpu.sync_copy(pe2_sh, rows_v.at[b])

    def start_gather(g, b):
        for p in range(_PPC):
            pltpu.async_copy(
                table_hbm.at[idx_v.at[g * _PPC + p]],
                rows_v.at[b].at[pl.ds(p * _IPG, _IPG)],
                gsem.at[b],
                add=True,
            )

    def wait_gather(b):
        pltpu.make_async_copy(
            out_hbm.at[pl.ds(0, _C)], rows_v.at[b], gsem.at[b]
        ).wait()

    def start_out(g, b):
        pltpu.async_copy(
            rows_v.at[b], out_hbm.at[pl.ds(row0 + g * _C, _C)], osem.at[b]
        )

    def wait_out(b):
        pltpu.make_async_copy(
            rows_v.at[b], out_hbm.at[pl.ds(0, _C)], osem.at[b]
        ).wait()

    prefill(0)
    start_gather(0, 0)

    def body(g, carry):
        b = g % 2
        nb = 1 - b

        @pl.when(g + 1 < _NG)
        def _():
            @pl.when(g >= 1)
            def _():
                wait_out(nb)
            prefill(nb)
            start_gather(g + 1, nb)

        wait_gather(b)
        start_out(g, b)
        return carry

    lax.fori_loop(0, _NG, body, 0)
    wait_out(0)
    wait_out(1)


def kernel(tokens, table):
    tok = tokens.reshape(-1).astype(jnp.int32).reshape(_B * _L // _IPG, _IPG)
    pe = _make_pe()
    out = _sc_embed(tok, pe, table)
    return out.reshape(_B, _L, _D)
